# Initial kernel scaffold; baseline (speedup 1.0000x reference)
import jax
import jax.numpy as jnp
from jax.experimental import pallas as pl


def kernel(embeddings):
    raise NotImplementedError("write your pallas kernel here")


# speedup vs baseline: 5.5482x; 5.5482x over previous
import functools

import jax
import jax.numpy as jnp
from jax import lax
from jax.experimental import pallas as pl
from jax.experimental.pallas import tpu as pltpu
from jax.experimental.pallas import tpu_sc as plsc

N = 4096
D = 128
K = 5
MARGIN = 1.0
EPS = 1e-12

RB = 256

NC = 2
NS = 16
NW = NC * NS
ROWS_PER_W = N // NW
RBLK = 8


def _dist_body(xb_ref, xf_ref, out_ref):
    xb = xb_ref[...]
    xf = xf_ref[...]
    x2b = jnp.sum(xb * xb, axis=1, keepdims=True)
    x2f = jnp.sum(xf * xf, axis=1)[None, :]
    prod = lax.dot_general(
        xb, xf, (((1,), (1,)), ((), ())),
        preferred_element_type=jnp.float32,
        precision=lax.Precision.HIGHEST)
    d2 = jnp.maximum(x2b + x2f - 2.0 * prod, 0.0)
    i = pl.program_id(0)
    rows = i * RB + lax.broadcasted_iota(jnp.int32, (RB, N), 0)
    cols = lax.broadcasted_iota(jnp.int32, (RB, N), 1)
    out_ref[...] = jnp.where(rows == cols, jnp.inf, d2)


def _dist(x):
    return pl.pallas_call(
        _dist_body,
        grid=(N // RB,),
        in_specs=[
            pl.BlockSpec((RB, D), lambda i: (i, 0)),
            pl.BlockSpec((N, D), lambda i: (0, 0)),
        ],
        out_specs=pl.BlockSpec((RB, N), lambda i: (i, 0)),
        out_shape=jax.ShapeDtypeStruct((N, N), jnp.float32),
    )(x, x)


def _newton_sqrt(x):
    bits = plsc.bitcast(x, jnp.int32)
    y = plsc.bitcast((bits >> 1) + 0x1FBD1DF5, jnp.float32)
    for _ in range(3):
        y = 0.5 * (y + x / y)
    return y


def _sc_body(d_hbm, out_hbm, rows_v, out_v):
    cid = lax.axis_index("c")
    sid = lax.axis_index("s")
    wid = sid * NC + cid
    base = wid * ROWS_PER_W
    lane = lax.iota(jnp.int32, 16)
    inf16 = jnp.full((16,), jnp.inf, jnp.float32)

    def row_body(r, acc):
        def chunk_body(c, ms):
            m0, m1, m2, m3, m4 = ms
            v = rows_v[r, pl.ds(c * 16, 16)]
            lo = jnp.minimum(m0, v); v = jnp.maximum(m0, v); m0 = lo
            lo = jnp.minimum(m1, v); v = jnp.maximum(m1, v); m1 = lo
            lo = jnp.minimum(m2, v); v = jnp.maximum(m2, v); m2 = lo
            lo = jnp.minimum(m3, v); v = jnp.maximum(m3, v); m3 = lo
            m4 = jnp.minimum(m4, v)
            return (m0, m1, m2, m3, m4)

        ms = lax.fori_loop(0, N // 16, chunk_body, (inf16,) * 5)
        c = plsc.sort_key_val(ms[0], ms[0])[0]
        for i in range(1, K):
            s = plsc.sort_key_val(ms[i], ms[i], descending=True)[0]
            m = jnp.minimum(c, s)
            c = plsc.sort_key_val(m, m)[0]
        dist = _newton_sqrt(jnp.maximum(c, EPS))
        total = jnp.sum(jnp.where(lane < K, dist, 0.0))
        t = total * (1.0 / K) - MARGIN
        return acc + t * t

    def block_body(b, acc):
        pltpu.sync_copy(d_hbm.at[pl.ds(base + b * RBLK, RBLK)], rows_v)
        return lax.fori_loop(0, RBLK, row_body, acc)

    acc = lax.fori_loop(0, ROWS_PER_W // RBLK, block_body, 0.0)
    out_v[...] = jnp.where(lane == 0, acc, 0.0)
    pltpu.sync_copy(out_v, out_hbm.at[wid])


_sc_topk = functools.partial(
    pl.kernel,
    out_type=jax.ShapeDtypeStruct((NW, 16), jnp.float32),
    mesh=plsc.VectorSubcoreMesh(core_axis_name="c", subcore_axis_name="s"),
    scratch_types=[
        pltpu.VMEM((RBLK, N), jnp.float32),
        pltpu.VMEM((16,), jnp.float32),
    ],
    compiler_params=pltpu.CompilerParams(needs_layout_passes=False),
)(_sc_body)


def kernel(embeddings):
    d2 = _dist(embeddings)
    partials = _sc_topk(d2)
    return jnp.sum(partials) / N


# speedup vs baseline: 11.5715x; 2.0856x over previous
import functools

import jax
import jax.numpy as jnp
from jax import lax
from jax.experimental import pallas as pl
from jax.experimental.pallas import tpu as pltpu
from jax.experimental.pallas import tpu_sc as plsc

N = 4096
D = 128
K = 5
MARGIN = 1.0
EPS = 1e-12

RB = 256

NC = 2
NS = 16
NW = NC * NS
ROWS_PER_W = N // NW
RBLK = 8


def _dist_body(xb_ref, xf_ref, out_ref):
    xb = xb_ref[...]
    xf = xf_ref[...]
    x2b = jnp.sum(xb * xb, axis=1, keepdims=True)
    x2f = jnp.sum(xf * xf, axis=1)[None, :]
    prod = lax.dot_general(
        xb, xf, (((1,), (1,)), ((), ())),
        preferred_element_type=jnp.float32)
    d2 = jnp.maximum(x2b + x2f - 2.0 * prod, 0.0)
    i = pl.program_id(0)
    rows = i * RB + lax.broadcasted_iota(jnp.int32, (RB, N), 0)
    cols = lax.broadcasted_iota(jnp.int32, (RB, N), 1)
    out_ref[...] = jnp.where(rows == cols, jnp.inf, d2)


def _dist(x):
    return pl.pallas_call(
        _dist_body,
        grid=(N // RB,),
        in_specs=[
            pl.BlockSpec((RB, D), lambda i: (i, 0)),
            pl.BlockSpec((N, D), lambda i: (0, 0)),
        ],
        out_specs=pl.BlockSpec((RB, N), lambda i: (i, 0)),
        out_shape=jax.ShapeDtypeStruct((N, N), jnp.float32),
    )(x, x)


def _newton_sqrt(x):
    bits = plsc.bitcast(x, jnp.int32)
    y = plsc.bitcast((bits >> 1) + 0x1FBD1DF5, jnp.float32)
    for _ in range(3):
        y = 0.5 * (y + x / y)
    return y


UNROLL = 8
NBLK = ROWS_PER_W // RBLK


def _sc_body(d_hbm, out_hbm, rows_v, out_v, sem0, sem1):
    cid = lax.axis_index("c")
    sid = lax.axis_index("s")
    wid = sid * NC + cid
    base = wid * ROWS_PER_W
    lane = lax.iota(jnp.int32, 16)
    inf16 = jnp.full((16,), jnp.inf, jnp.float32)
    sems = (sem0, sem1)

    def row_body(slot):
        def body(r, acc):
            def chunk_body(c, ms):
                m0, m1, m2, m3, m4 = ms
                cb = c * (16 * UNROLL)
                for u in range(UNROLL):
                    v = rows_v[slot, r, pl.ds(cb + u * 16, 16)]
                    lo = jnp.minimum(m0, v); v = jnp.maximum(m0, v); m0 = lo
                    lo = jnp.minimum(m1, v); v = jnp.maximum(m1, v); m1 = lo
                    lo = jnp.minimum(m2, v); v = jnp.maximum(m2, v); m2 = lo
                    lo = jnp.minimum(m3, v); v = jnp.maximum(m3, v); m3 = lo
                    m4 = jnp.minimum(m4, v)
                return (m0, m1, m2, m3, m4)

            ms = lax.fori_loop(0, N // (16 * UNROLL), chunk_body,
                               (inf16,) * 5)
            c = plsc.sort_key_val(ms[0], ms[0])[0]
            for i in range(1, K):
                s = plsc.sort_key_val(ms[i], ms[i], descending=True)[0]
                m = jnp.minimum(c, s)
                c = plsc.sort_key_val(m, m)[0]
            dist = _newton_sqrt(jnp.maximum(c, EPS))
            total = jnp.sum(jnp.where(lane < K, dist, 0.0))
            t = total * (1.0 / K) - MARGIN
            return acc + t * t

        return body

    copies = [
        pltpu.async_copy(d_hbm.at[pl.ds(base + b * RBLK, RBLK)],
                         rows_v.at[b], sems[b])
        for b in range(2)
    ]
    acc = 0.0
    for b in range(NBLK):
        slot = b % 2
        copies[slot].wait()
        acc = lax.fori_loop(0, RBLK, row_body(slot), acc)
        if b + 2 < NBLK:
            copies[slot] = pltpu.async_copy(
                d_hbm.at[pl.ds(base + (b + 2) * RBLK, RBLK)],
                rows_v.at[slot], sems[slot])
    out_v[...] = jnp.where(lane == 0, acc, 0.0)
    pltpu.sync_copy(out_v, out_hbm.at[wid])


_sc_topk = functools.partial(
    pl.kernel,
    out_type=jax.ShapeDtypeStruct((NW, 16), jnp.float32),
    mesh=plsc.VectorSubcoreMesh(core_axis_name="c", subcore_axis_name="s"),
    scratch_types=[
        pltpu.VMEM((2, RBLK, N), jnp.float32),
        pltpu.VMEM((16,), jnp.float32),
        pltpu.SemaphoreType.DMA,
        pltpu.SemaphoreType.DMA,
    ],
    compiler_params=pltpu.CompilerParams(needs_layout_passes=False),
)(_sc_body)


def kernel(embeddings):
    d2 = _dist(embeddings)
    partials = _sc_topk(d2)
    return jnp.sum(partials) / N
